# Initial kernel scaffold; baseline (speedup 1.0000x reference)
import jax
import jax.numpy as jnp
from jax.experimental import pallas as pl


def kernel(x, edge_index, W_rel1, b_rel1, W_root1, a1, W_rel2, b_rel2, W_root2, a2, W_post, b_post):
    raise NotImplementedError("write your pallas kernel here")


# speedup vs baseline: 9.0826x; 9.0826x over previous
import functools

import jax
import jax.numpy as jnp
from jax import lax
from jax.experimental import pallas as pl
from jax.experimental.pallas import tpu as pltpu
from jax.experimental.pallas import tpu_sc as plsc

N = 10000
E = 320000
D = 128
H = 32

NC = 2
NS = 16
NW = NC * NS
EPW = E // NW
B = 80
NGRP = EPW // B
RPT = N // NS

_f32 = jnp.float32


def _seg_body(msg_h, src_h, dst_h, zeros_h, out_h, acc, src_v, dst_v, rows_v, sem):
    c = lax.axis_index("c")
    s = lax.axis_index("s")
    wid = c * NS + s

    pltpu.sync_copy(zeros_h.at[pl.ds(s * RPT, RPT)], acc.at[pl.ds(s * RPT, RPT)])
    pltpu.sync_copy(src_h.at[wid], src_v)
    pltpu.sync_copy(dst_h.at[wid], dst_v)
    plsc.subcore_barrier()

    def step(g, carry):
        pltpu.async_copy(msg_h.at[src_v.at[g]], rows_v, sem).wait()
        pltpu.sync_copy(rows_v, acc.at[dst_v.at[g]], add=True)
        return carry

    lax.fori_loop(0, NGRP, step, 0)
    plsc.subcore_barrier()
    pltpu.sync_copy(acc.at[pl.ds(s * RPT, RPT)], out_h.at[c, pl.ds(s * RPT, RPT)])


_segsum = pl.kernel(
    _seg_body,
    out_type=jax.ShapeDtypeStruct((NC, N, H), _f32),
    mesh=plsc.VectorSubcoreMesh(core_axis_name="c", subcore_axis_name="s"),
    compiler_params=pltpu.CompilerParams(use_tc_tiling_on_sc=False),
    scratch_types=[
        pltpu.VMEM_SHARED((N, H), _f32),
        pltpu.VMEM((NGRP, B), jnp.int32),
        pltpu.VMEM((NGRP, B), jnp.int32),
        pltpu.VMEM((B, H), _f32),
        pltpu.SemaphoreType.DMA,
    ],
)


def _tc1_body(x_ref, wrel_ref, wroot_ref, y_ref, r_ref):
    xv = x_ref[...]
    y_ref[...] = jnp.dot(xv, wrel_ref[...], preferred_element_type=_f32, precision=jax.lax.Precision.HIGHEST)
    r_ref[...] = jnp.dot(xv, wroot_ref[...], preferred_element_type=_f32, precision=jax.lax.Precision.HIGHEST)


def _tc2_body(p_ref, r_ref, b_ref, a_ref, wrel_ref, wroot_ref, y2_ref, r2_ref):
    pre = p_ref[0] + p_ref[1] + b_ref[...] + r_ref[...]
    h = jnp.where(pre > 0, pre, a_ref[...] * pre)
    y2_ref[...] = jnp.dot(h, wrel_ref[...], preferred_element_type=_f32, precision=jax.lax.Precision.HIGHEST)
    r2_ref[...] = jnp.dot(h, wroot_ref[...], preferred_element_type=_f32, precision=jax.lax.Precision.HIGHEST)


def _tc3_body(p_ref, r2_ref, b_ref, a_ref, wpost_ref, bpost_ref, o_ref):
    pre = p_ref[0] + p_ref[1] + b_ref[...] + r2_ref[...]
    h2 = jnp.where(pre > 0, pre, a_ref[...] * pre)
    o_ref[...] = jnp.dot(h2, wpost_ref[...], preferred_element_type=_f32, precision=jax.lax.Precision.HIGHEST) + bpost_ref[...]


_sds = jax.ShapeDtypeStruct
_tc1 = pl.pallas_call(_tc1_body, out_shape=(_sds((N, H), _f32), _sds((N, H), _f32)))
_tc2 = pl.pallas_call(_tc2_body, out_shape=(_sds((N, H), _f32), _sds((N, H), _f32)))
_tc3 = pl.pallas_call(_tc3_body, out_shape=_sds((N, 1), _f32))


def kernel(x, edge_index, W_rel1, b_rel1, W_root1, a1, W_rel2, b_rel2, W_root2, a2, W_post, b_post):
    src = edge_index[0].astype(jnp.int32).reshape(NW, NGRP, B)
    dst = edge_index[1].astype(jnp.int32).reshape(NW, NGRP, B)
    zeros = jnp.zeros((N, H), _f32)
    b1 = b_rel1.reshape(1, H)
    b2 = b_rel2.reshape(1, H)
    a1v = a1.reshape(1, 1)
    a2v = a2.reshape(1, 1)
    bp = b_post.reshape(1, 1)

    y1, r1 = _tc1(x, W_rel1, W_root1)
    p1 = _segsum(y1, src, dst, zeros)
    y2, r2 = _tc2(p1, r1, b1, a1v, W_rel2, W_root2)
    p2 = _segsum(y2, src, dst, zeros)
    return _tc3(p2, r2, b2, a2v, W_post, bp)


# speedup vs baseline: 15.4479x; 1.7008x over previous
import functools

import jax
import jax.numpy as jnp
from jax import lax
from jax.experimental import pallas as pl
from jax.experimental.pallas import tpu as pltpu
from jax.experimental.pallas import tpu_sc as plsc

N = 10000
E = 320000
D = 128
H = 32

NC = 2
NS = 16
NW = NC * NS
EPW = E // NW
B = 80
NGRP = EPW // B
RPT = N // NS

_f32 = jnp.float32


KB = 5
BB = NGRP // KB


def _seg_body(msg_h, src_h, dst_h, zeros_h, out_h, acc, msg_s, src_v, dst_v,
              rows_v, sems):
    c = lax.axis_index("c")
    s = lax.axis_index("s")
    wid = c * NS + s

    pltpu.sync_copy(zeros_h.at[pl.ds(s * RPT, RPT)], acc.at[pl.ds(s * RPT, RPT)])
    pltpu.sync_copy(msg_h.at[pl.ds(s * RPT, RPT)], msg_s.at[pl.ds(s * RPT, RPT)])
    pltpu.sync_copy(src_h.at[wid], src_v)
    pltpu.sync_copy(dst_h.at[wid], dst_v)
    plsc.subcore_barrier()

    def fire(bb, buf):
        for j in range(KB):
            pltpu.async_copy(msg_s.at[src_v.at[bb * KB + j]],
                             rows_v.at[buf, pl.ds(j * B, B)], sems.at[buf])

    def drain_scatter(bb, buf):
        for j in range(KB):
            pltpu.make_async_copy(msg_s.at[src_v.at[bb * KB + j]],
                                  rows_v.at[buf, pl.ds(j * B, B)],
                                  sems.at[buf]).wait()
        for j in range(KB):
            pltpu.sync_copy(rows_v.at[buf, pl.ds(j * B, B)],
                            acc.at[dst_v.at[bb * KB + j]], add=True)

    fire(0, 0)

    def step(bb, carry):
        buf = lax.rem(bb, 2)

        @pl.when(bb + 1 < BB)
        def _():
            fire(bb + 1, 1 - buf)

        drain_scatter(bb, buf)
        return carry

    lax.fori_loop(0, BB, step, 0)
    plsc.subcore_barrier()
    pltpu.sync_copy(acc.at[pl.ds(s * RPT, RPT)], out_h.at[c, pl.ds(s * RPT, RPT)])


_segsum = pl.kernel(
    _seg_body,
    out_type=jax.ShapeDtypeStruct((NC, N, H), _f32),
    mesh=plsc.VectorSubcoreMesh(core_axis_name="c", subcore_axis_name="s"),
    compiler_params=pltpu.CompilerParams(use_tc_tiling_on_sc=False),
    scratch_types=[
        pltpu.VMEM_SHARED((N, H), _f32),
        pltpu.VMEM_SHARED((N, H), _f32),
        pltpu.VMEM((NGRP, B), jnp.int32),
        pltpu.VMEM((NGRP, B), jnp.int32),
        pltpu.VMEM((2, KB * B, H), _f32),
        pltpu.SemaphoreType.DMA((2,)),
    ],
)


def _tc1_body(x_ref, wrel_ref, wroot_ref, y_ref, r_ref):
    xv = x_ref[...]
    y_ref[...] = jnp.dot(xv, wrel_ref[...], preferred_element_type=_f32, precision=jax.lax.Precision.HIGHEST)
    r_ref[...] = jnp.dot(xv, wroot_ref[...], preferred_element_type=_f32, precision=jax.lax.Precision.HIGHEST)


def _tc2_body(p_ref, r_ref, b_ref, a_ref, wrel_ref, wroot_ref, y2_ref, r2_ref):
    pre = p_ref[0] + p_ref[1] + b_ref[...] + r_ref[...]
    h = jnp.where(pre > 0, pre, a_ref[...] * pre)
    y2_ref[...] = jnp.dot(h, wrel_ref[...], preferred_element_type=_f32, precision=jax.lax.Precision.HIGHEST)
    r2_ref[...] = jnp.dot(h, wroot_ref[...], preferred_element_type=_f32, precision=jax.lax.Precision.HIGHEST)


def _tc3_body(p_ref, r2_ref, b_ref, a_ref, wpost_ref, bpost_ref, o_ref):
    pre = p_ref[0] + p_ref[1] + b_ref[...] + r2_ref[...]
    h2 = jnp.where(pre > 0, pre, a_ref[...] * pre)
    o_ref[...] = jnp.dot(h2, wpost_ref[...], preferred_element_type=_f32, precision=jax.lax.Precision.HIGHEST) + bpost_ref[...]


_sds = jax.ShapeDtypeStruct
_tc1 = pl.pallas_call(_tc1_body, out_shape=(_sds((N, H), _f32), _sds((N, H), _f32)))
_tc2 = pl.pallas_call(_tc2_body, out_shape=(_sds((N, H), _f32), _sds((N, H), _f32)))
_tc3 = pl.pallas_call(_tc3_body, out_shape=_sds((N, 1), _f32))


def kernel(x, edge_index, W_rel1, b_rel1, W_root1, a1, W_rel2, b_rel2, W_root2, a2, W_post, b_post):
    src = edge_index[0].astype(jnp.int32).reshape(NW, NGRP, B)
    dst = edge_index[1].astype(jnp.int32).reshape(NW, NGRP, B)
    zeros = jnp.zeros((N, H), _f32)
    b1 = b_rel1.reshape(1, H)
    b2 = b_rel2.reshape(1, H)
    a1v = a1.reshape(1, 1)
    a2v = a2.reshape(1, 1)
    bp = b_post.reshape(1, 1)

    y1, r1 = _tc1(x, W_rel1, W_root1)
    p1 = _segsum(y1, src, dst, zeros)
    y2, r2 = _tc2(p1, r1, b1, a1v, W_rel2, W_root2)
    p2 = _segsum(y2, src, dst, zeros)
    return _tc3(p2, r2, b2, a2v, W_post, bp)


# speedup vs baseline: 23.6129x; 1.5285x over previous
import functools

import jax
import jax.numpy as jnp
from jax import lax
from jax.experimental import pallas as pl
from jax.experimental.pallas import tpu as pltpu
from jax.experimental.pallas import tpu_sc as plsc

N = 10000
E = 320000
D = 128
H = 32

NC = 2
NS = 16
NW = NC * NS
B = 128
NROW = E // B
RPB = NROW // NW
REM = NROW % NW
RPT = N // NS
NBUF = 4

_f32 = jnp.float32


def _seg_body(msg_h, src_h, dst_h, out0_h, out1_h, acc, src_v, dst_v, rows_v, sems):
    c = lax.axis_index("c")
    s = lax.axis_index("s")
    wid = c * NS + s
    n_ops = RPB + jnp.where(wid < REM, 1, 0)
    row_base = RPB * wid + jnp.minimum(wid, REM)

    zero16 = jnp.zeros((16,), _f32)

    def zrow(i, carry):
        rows_v[NBUF - 1, i, pl.ds(0, 16)] = zero16
        rows_v[NBUF - 1, i, pl.ds(16, 16)] = zero16
        return carry

    lax.fori_loop(0, B, zrow, 0)
    for k in range(RPT // B):
        pltpu.sync_copy(rows_v.at[NBUF - 1, pl.ds(0, B)],
                        acc.at[pl.ds(s * RPT + k * B, B)])
    if RPT % B:
        pltpu.sync_copy(rows_v.at[NBUF - 1, pl.ds(0, RPT % B)],
                        acc.at[pl.ds(s * RPT + (RPT // B) * B, RPT % B)])

    @pl.when(wid < REM)
    def _():
        pltpu.sync_copy(src_h.at[pl.ds(row_base, RPB + 1)], src_v)
        pltpu.sync_copy(dst_h.at[pl.ds(row_base, RPB + 1)], dst_v)

    @pl.when(wid >= REM)
    def _():
        pltpu.sync_copy(src_h.at[pl.ds(row_base, RPB)], src_v.at[pl.ds(0, RPB)])
        pltpu.sync_copy(dst_h.at[pl.ds(row_base, RPB)], dst_v.at[pl.ds(0, RPB)])

    plsc.subcore_barrier()

    def fire(g):
        buf = lax.rem(g, NBUF)
        pltpu.async_copy(msg_h.at[src_v.at[g]], rows_v.at[buf], sems.at[buf])

    for g0 in range(NBUF - 1):
        fire(jnp.int32(g0))

    def step(g, carry):
        @pl.when(g + (NBUF - 1) < n_ops)
        def _():
            fire(g + (NBUF - 1))

        buf = lax.rem(g, NBUF)
        pltpu.make_async_copy(msg_h.at[src_v.at[g]], rows_v.at[buf],
                              sems.at[buf]).wait()
        pltpu.sync_copy(rows_v.at[buf], acc.at[dst_v.at[g]], add=True)
        return carry

    lax.fori_loop(0, n_ops, step, 0)
    plsc.subcore_barrier()

    @pl.when(c == 0)
    def _():
        pltpu.sync_copy(acc.at[pl.ds(s * RPT, RPT)], out0_h.at[pl.ds(s * RPT, RPT)])

    @pl.when(c == 1)
    def _():
        pltpu.sync_copy(acc.at[pl.ds(s * RPT, RPT)], out1_h.at[pl.ds(s * RPT, RPT)])


_segsum = pl.kernel(
    _seg_body,
    out_type=(jax.ShapeDtypeStruct((N, H), _f32),
              jax.ShapeDtypeStruct((N, H), _f32)),
    mesh=plsc.VectorSubcoreMesh(core_axis_name="c", subcore_axis_name="s"),
    compiler_params=pltpu.CompilerParams(use_tc_tiling_on_sc=False),
    scratch_types=[
        pltpu.VMEM_SHARED((N, H), _f32),
        pltpu.VMEM((RPB + 1, B), jnp.int32),
        pltpu.VMEM((RPB + 1, B), jnp.int32),
        pltpu.VMEM((NBUF, B, H), _f32),
        pltpu.SemaphoreType.DMA((NBUF,)),
    ],
)


PK = 4
NP = N // PK
MBP = 256
_HI = jax.lax.Precision.HIGHEST


def _tc1_body(x_ref, wrel_ref, wroot_ref, y_ref, r_ref):
    xv = x_ref[...]
    ys = [jnp.dot(xv[:, q, :], wrel_ref[...], preferred_element_type=_f32, precision=_HI)
          for q in range(PK)]
    rs = [jnp.dot(xv[:, q, :], wroot_ref[...], preferred_element_type=_f32, precision=_HI)
          for q in range(PK)]
    y_ref[...] = jnp.concatenate(ys, axis=1)
    r_ref[...] = jnp.concatenate(rs, axis=1)


def _tc2_body(p0_ref, p1_ref, r_ref, b_ref, a_ref, wrel_ref, wroot_ref, y2_ref, r2_ref):
    pre = p0_ref[...] + p1_ref[...] + b_ref[...] + r_ref[...]
    h = jnp.where(pre > 0, pre, a_ref[...] * pre)
    y2_ref[...] = jnp.dot(h, wrel_ref[...], preferred_element_type=_f32, precision=_HI)
    r2_ref[...] = jnp.dot(h, wroot_ref[...], preferred_element_type=_f32, precision=_HI)


def _tc3_body(p0_ref, p1_ref, r2_ref, b_ref, a_ref, wpost_ref, bpost_ref, o_ref):
    pre = p0_ref[...] + p1_ref[...] + b_ref[...] + r2_ref[...]
    h2 = jnp.where(pre > 0, pre, a_ref[...] * pre)
    o_ref[...] = jnp.dot(h2, wpost_ref[...], preferred_element_type=_f32, precision=_HI) + bpost_ref[...]


_sds = jax.ShapeDtypeStruct
_full = lambda shape: pl.BlockSpec(shape, lambda i: (0,) * len(shape))
_rows = lambda shape: pl.BlockSpec((MBP,) + shape[1:], lambda i: (i,) + (0,) * (len(shape) - 1))
_prows = _rows((NP, PK * H))
_GRID = (NP + MBP - 1) // MBP

_tc1 = pl.pallas_call(
    _tc1_body,
    grid=(_GRID,),
    in_specs=[_rows((NP, PK, D)), _full((D, H)), _full((D, H))],
    out_specs=(_rows((NP, PK * H)), _rows((NP, PK * H))),
    out_shape=(_sds((NP, PK * H), _f32), _sds((NP, PK * H), _f32)),
)
_tc2 = pl.pallas_call(
    _tc2_body,
    grid=(_GRID,),
    in_specs=[_prows, _prows, _rows((NP, PK * H)), _full((1, PK * H)),
              _full((1, 1)), _full((PK * H, PK * H)), _full((PK * H, PK * H))],
    out_specs=(_rows((NP, PK * H)), _rows((NP, PK * H))),
    out_shape=(_sds((NP, PK * H), _f32), _sds((NP, PK * H), _f32)),
)
_tc3 = pl.pallas_call(
    _tc3_body,
    grid=(_GRID,),
    in_specs=[_prows, _prows, _rows((NP, PK * H)), _full((1, PK * H)),
              _full((1, 1)), _full((PK * H, PK)), _full((1, PK))],
    out_specs=_rows((NP, PK)),
    out_shape=_sds((NP, PK), _f32),
)


def _blkdiag(w):
    eye = jnp.eye(PK, dtype=w.dtype)
    return jnp.einsum("ab,km->akbm", eye, w).reshape(PK * H, PK * w.shape[1])


def kernel(x, edge_index, W_rel1, b_rel1, W_root1, a1, W_rel2, b_rel2, W_root2, a2, W_post, b_post):
    src = edge_index[0].astype(jnp.int32).reshape(NROW, B)
    dst = edge_index[1].astype(jnp.int32).reshape(NROW, B)
    x4 = x.reshape(NP, PK, D)
    b1 = jnp.tile(b_rel1, PK).reshape(1, PK * H)
    b2 = jnp.tile(b_rel2, PK).reshape(1, PK * H)
    a1v = a1.reshape(1, 1)
    a2v = a2.reshape(1, 1)
    bp = jnp.tile(b_post, PK).reshape(1, PK)
    wrel2 = _blkdiag(W_rel2)
    wroot2 = _blkdiag(W_root2)
    wpost = _blkdiag(W_post)

    y1p, r1p = _tc1(x4, W_rel1, W_root1)
    p1a, p1b = _segsum(y1p.reshape(N, H), src, dst)
    y2p, r2p = _tc2(p1a.reshape(NP, PK * H), p1b.reshape(NP, PK * H),
                    r1p, b1, a1v, wrel2, wroot2)
    p2a, p2b = _segsum(y2p.reshape(N, H), src, dst)
    outp = _tc3(p2a.reshape(NP, PK * H), p2b.reshape(NP, PK * H),
                r2p, b2, a2v, wpost, bp)
    return outp.reshape(N, 1)
